# baseline (speedup 1.0000x reference)
import functools

import jax
import jax.numpy as jnp
from jax import lax
from jax.experimental import pallas as pl
from jax.experimental.pallas import tpu as pltpu
from jax.experimental.pallas import tpu_sc as plsc

_L = 16


def _build_permute(num_batches, rows, nh, num_scopes, num_decomps):
    num_workers = 32
    batches_per_worker = num_batches // num_workers
    half = rows // 2
    scopes_per_half = num_scopes // 2
    mesh = plsc.VectorSubcoreMesh(core_axis_name="c", subcore_axis_name="s",
                                  num_cores=2, num_subcores=16)
    nc = mesh.num_cores
    perm_groups = (num_scopes * num_decomps) // _L
    row_stride = 2 * num_decomps

    @functools.partial(
        pl.kernel,
        out_type=jax.ShapeDtypeStruct((num_batches, rows, nh), jnp.float32),
        mesh=mesh,
        scratch_types=[
            pltpu.VMEM((2, rows + row_stride, nh), jnp.float32),
            pltpu.VMEM((2, half, nh), jnp.float32),
            pltpu.VMEM((num_scopes * num_decomps,), jnp.int32),
            pltpu.SMEM((num_scopes * num_decomps,), jnp.int32),
            pltpu.SemaphoreType.DMA,
            pltpu.SemaphoreType.DMA,
            pltpu.SemaphoreType.DMA,
            pltpu.SemaphoreType.DMA,
        ],
    )
    def permute_kernel(x_hbm, perm_hbm, out_hbm, abuf, sbuf, perm_v, pes,
                      gsem0, gsem1, psem0, psem1):
        wid = lax.axis_index("s") * nc + lax.axis_index("c")
        base_b = wid * batches_per_worker

        pltpu.sync_copy(perm_hbm, perm_v)
        for j in range(perm_groups):
            pv = perm_v[pl.ds(_L * j, _L)]
            for l in range(_L):
                p = pv[l]
                psel = jnp.where(p < 0, num_scopes, p)
                d_static = (_L * j + l) % num_decomps
                pes[_L * j + l] = row_stride * psel + d_static

        zvec = jnp.zeros((_L,), jnp.float32)
        for slot in range(2):
            for r in range(row_stride):
                for k in range(nh // _L):
                    abuf[slot, rows + r, pl.ds(_L * k, _L)] = zvec

        gsems = (gsem0, gsem1)
        psems = (psem0, psem1)

        def gather_cp(i, slot):
            return pltpu.make_async_copy(
                x_hbm.at[base_b + i], abuf.at[slot, pl.ds(0, rows)],
                gsems[slot])

        def scatter_cp(i, h):
            return pltpu.make_async_copy(
                sbuf.at[h], out_hbm.at[base_b + i, pl.ds(half * h, half)],
                psems[h])

        def shuffle_half(slot, h):
            @plsc.parallel_loop(0, scopes_per_half, unroll=4)
            def _scope(sl):
                sg = scopes_per_half * h + sl
                for d in range(num_decomps):
                    base = pes[sg * num_decomps + d]
                    for c in range(2):
                        usrc = base + num_decomps * c
                        udst = row_stride * sl + num_decomps * c + d
                        for k in range(nh // _L):
                            sbuf[h, udst, pl.ds(_L * k, _L)] = (
                                abuf[slot, usrc, pl.ds(_L * k, _L)])

        gather_cp(0, 0).start()

        @pl.loop(0, batches_per_worker, step=2)
        def _batch(i):
            for sub in range(2):
                jj = i + sub
                slot = sub

                @pl.when(jj + 1 < batches_per_worker)
                def _prefetch():
                    gather_cp(jj + 1, 1 - slot).start()

                gather_cp(jj, slot).wait()
                for h in range(2):
                    @pl.when(jj > 0)
                    def _drain():
                        scatter_cp(jj - 1, h).wait()

                    shuffle_half(slot, h)
                    scatter_cp(jj, h).start()

        for h in range(2):
            scatter_cp(batches_per_worker - 1, h).wait()

    return permute_kernel


def kernel(x, permutations):
    b, s, d, n = x.shape
    nh = n // 2
    rows = s * 2 * d
    xr = (x.reshape(b, s, d, 2, nh).transpose(0, 1, 3, 2, 4)
          .reshape(b, rows, nh))
    permf = permutations.T.reshape(d * s)
    permute = _build_permute(b, rows, nh, s, d)
    out = permute(xr, permf)
    return (out.reshape(b, s, 2, d, nh).transpose(0, 1, 3, 2, 4)
            .reshape(b, s, d, n))


# speedup vs baseline: 1.1457x; 1.1457x over previous
import functools

import jax
import jax.numpy as jnp
from jax import lax
from jax.experimental import pallas as pl
from jax.experimental.pallas import tpu as pltpu
from jax.experimental.pallas import tpu_sc as plsc

_L = 16


def _build_permute(num_batches, rows, nh, num_scopes, num_decomps):
    num_workers = 32
    batches_per_worker = num_batches // num_workers
    half = rows // 2
    scopes_per_half = num_scopes // 2
    mesh = plsc.VectorSubcoreMesh(core_axis_name="c", subcore_axis_name="s",
                                  num_cores=2, num_subcores=16)
    nc = mesh.num_cores
    perm_groups = (num_scopes * num_decomps) // _L
    row_stride = 2 * num_decomps

    @functools.partial(
        pl.kernel,
        out_type=jax.ShapeDtypeStruct((num_batches, rows, nh), jnp.float32),
        mesh=mesh,
        scratch_types=[
            pltpu.VMEM((2, rows + row_stride, nh), jnp.float32),
            pltpu.VMEM((2, half, nh), jnp.float32),
            pltpu.VMEM((num_scopes * num_decomps,), jnp.int32),
            pltpu.SMEM((num_scopes * num_decomps,), jnp.int32),
            pltpu.SemaphoreType.DMA,
            pltpu.SemaphoreType.DMA,
            pltpu.SemaphoreType.DMA,
            pltpu.SemaphoreType.DMA,
        ],
    )
    def permute_kernel(x_hbm, perm_hbm, out_hbm, abuf, sbuf, perm_v, pes,
                      gsem0, gsem1, psem0, psem1):
        wid = lax.axis_index("s") * nc + lax.axis_index("c")
        base_b = wid * batches_per_worker

        pltpu.sync_copy(perm_hbm, perm_v)
        for j in range(perm_groups):
            pv = perm_v[pl.ds(_L * j, _L)]
            for l in range(_L):
                p = pv[l]
                psel = jnp.where(p < 0, num_scopes, p)
                d_static = (_L * j + l) % num_decomps
                pes[_L * j + l] = row_stride * psel + d_static

        zvec = jnp.zeros((_L,), jnp.float32)
        for slot in range(2):
            for r in range(row_stride):
                for k in range(nh // _L):
                    abuf[slot, rows + r, pl.ds(_L * k, _L)] = zvec

        gsems = (gsem0, gsem1)
        psems = (psem0, psem1)

        def gather_cp(i, slot):
            return pltpu.make_async_copy(
                x_hbm.at[base_b + i], abuf.at[slot, pl.ds(0, rows)],
                gsems[slot])

        def scatter_cp(i, h):
            return pltpu.make_async_copy(
                sbuf.at[h], out_hbm.at[base_b + i, pl.ds(half * h, half)],
                psems[h])

        def shuffle_half(slot, h):
            @plsc.parallel_loop(0, scopes_per_half, unroll=2)
            def _scope(sl):
                sg = scopes_per_half * h + sl
                for d in range(num_decomps):
                    base = pes[sg * num_decomps + d]
                    for c in range(2):
                        usrc = base + num_decomps * c
                        udst = row_stride * sl + num_decomps * c + d
                        for k in range(nh // _L):
                            sbuf[h, udst, pl.ds(_L * k, _L)] = (
                                abuf[slot, usrc, pl.ds(_L * k, _L)])

        gather_cp(0, 0).start()

        @pl.loop(0, batches_per_worker, step=2)
        def _batch(i):
            for sub in range(2):
                jj = i + sub
                slot = sub

                @pl.when(jj + 1 < batches_per_worker)
                def _prefetch():
                    gather_cp(jj + 1, 1 - slot).start()

                gather_cp(jj, slot).wait()
                for h in range(2):
                    @pl.when(jj > 0)
                    def _drain():
                        scatter_cp(jj - 1, h).wait()

                    shuffle_half(slot, h)
                    scatter_cp(jj, h).start()

        for h in range(2):
            scatter_cp(batches_per_worker - 1, h).wait()

    return permute_kernel


def kernel(x, permutations):
    b, s, d, n = x.shape
    nh = n // 2
    rows = s * 2 * d
    xr = (x.reshape(b, s, d, 2, nh).transpose(0, 1, 3, 2, 4)
          .reshape(b, rows, nh))
    permf = permutations.T.reshape(d * s)
    permute = _build_permute(b, rows, nh, s, d)
    out = permute(xr, permf)
    return (out.reshape(b, s, 2, d, nh).transpose(0, 1, 3, 2, 4)
            .reshape(b, s, d, n))


# speedup vs baseline: 1.2363x; 1.0791x over previous
import functools

import jax
import jax.numpy as jnp
from jax import lax
from jax.experimental import pallas as pl
from jax.experimental.pallas import tpu as pltpu
from jax.experimental.pallas import tpu_sc as plsc

_L = 16


def _build_permute(num_batches, rows, nh, num_scopes, num_decomps):
    num_workers = 32
    batches_per_worker = num_batches // num_workers
    half = rows // 2
    scopes_per_half = num_scopes // 2
    mesh = plsc.VectorSubcoreMesh(core_axis_name="c", subcore_axis_name="s",
                                  num_cores=2, num_subcores=16)
    nc = mesh.num_cores
    perm_groups = (num_scopes * num_decomps) // _L
    row_stride = 2 * num_decomps

    @functools.partial(
        pl.kernel,
        out_type=jax.ShapeDtypeStruct((num_batches, rows, nh), jnp.float32),
        mesh=mesh,
        scratch_types=[
            pltpu.VMEM((2, rows + row_stride, nh), jnp.float32),
            pltpu.VMEM((2, half, nh), jnp.float32),
            pltpu.VMEM((num_scopes * num_decomps,), jnp.int32),
            pltpu.SMEM((num_scopes * num_decomps,), jnp.int32),
            pltpu.SemaphoreType.DMA,
            pltpu.SemaphoreType.DMA,
            pltpu.SemaphoreType.DMA,
            pltpu.SemaphoreType.DMA,
        ],
    )
    def permute_kernel(x_hbm, perm_hbm, out_hbm, abuf, sbuf, perm_v, pes,
                      gsem0, gsem1, psem0, psem1):
        wid = lax.axis_index("s") * nc + lax.axis_index("c")
        base_b = wid * batches_per_worker

        pltpu.sync_copy(perm_hbm, perm_v)
        for j in range(perm_groups):
            pv = perm_v[pl.ds(_L * j, _L)]
            for l in range(_L):
                p = pv[l]
                psel = jnp.where(p < 0, num_scopes, p)
                d_static = (_L * j + l) % num_decomps
                pes[_L * j + l] = row_stride * psel + d_static

        zvec = jnp.zeros((_L,), jnp.float32)
        for slot in range(2):
            for r in range(row_stride):
                for k in range(nh // _L):
                    abuf[slot, rows + r, pl.ds(_L * k, _L)] = zvec

        gsems = (gsem0, gsem1)
        psems = (psem0, psem1)

        def gather_cp(i, slot):
            return pltpu.make_async_copy(
                x_hbm.at[base_b + i], abuf.at[slot, pl.ds(0, rows)],
                gsems[slot])

        def scatter_cp(i, h):
            return pltpu.make_async_copy(
                sbuf.at[h], out_hbm.at[base_b + i, pl.ds(half * h, half)],
                psems[h])

        def shuffle_half(slot, h):
            half_off = scopes_per_half * num_decomps * h

            @plsc.parallel_loop(0, scopes_per_half * num_decomps, unroll=2)
            def _sd(i):
                base = pes[half_off + i]
                sl = i // num_decomps
                d = i - num_decomps * sl
                for c in range(2):
                    usrc = base + num_decomps * c
                    udst = row_stride * sl + num_decomps * c + d
                    for k in range(nh // _L):
                        sbuf[h, udst, pl.ds(_L * k, _L)] = (
                            abuf[slot, usrc, pl.ds(_L * k, _L)])

        gather_cp(0, 0).start()

        @pl.loop(0, batches_per_worker, step=2)
        def _batch(i):
            for sub in range(2):
                jj = i + sub
                slot = sub

                @pl.when(jj + 1 < batches_per_worker)
                def _prefetch():
                    gather_cp(jj + 1, 1 - slot).start()

                gather_cp(jj, slot).wait()
                for h in range(2):
                    @pl.when(jj > 0)
                    def _drain():
                        scatter_cp(jj - 1, h).wait()

                    shuffle_half(slot, h)
                    scatter_cp(jj, h).start()

        for h in range(2):
            scatter_cp(batches_per_worker - 1, h).wait()

    return permute_kernel


def kernel(x, permutations):
    b, s, d, n = x.shape
    nh = n // 2
    rows = s * 2 * d
    xr = (x.reshape(b, s, d, 2, nh).transpose(0, 1, 3, 2, 4)
          .reshape(b, rows, nh))
    permf = permutations.T.reshape(d * s)
    permute = _build_permute(b, rows, nh, s, d)
    out = permute(xr, permf)
    return (out.reshape(b, s, 2, d, nh).transpose(0, 1, 3, 2, 4)
            .reshape(b, s, d, n))
